# Initial kernel scaffold; baseline (speedup 1.0000x reference)
import jax
import jax.numpy as jnp
from jax.experimental import pallas as pl


def kernel(xyz_hr, xyz_lr, sft_feat_lr, val_lr, params):
    raise NotImplementedError("write your pallas kernel here")


# speedup vs baseline: 15.6190x; 15.6190x over previous
import functools
from typing import Any

import jax
import jax.numpy as jnp
from jax.experimental import pallas as pl
from jax.experimental.pallas import tpu as pltpu

KNN = 16
EPS = 1e-5


def _bn_stats(hs, count):
    s1 = sum(jnp.sum(h, axis=1, keepdims=True) for h in hs)
    s2 = sum(jnp.sum(h * h, axis=1, keepdims=True) for h in hs)
    m = s1 / count
    v = s2 / count - m * m
    return m, jax.lax.rsqrt(v + EPS)


def _p1_body(xyz_ref, sft_ref, w1_ref, b1_ref, g1_ref, bt1_ref, w2_ref,
             b2_ref, scw_ref, scb_ref, shw_ref, shb_ref, kw_ref, kb_ref,
             k_out_ref):
    B = xyz_ref.shape[0]
    M = xyz_ref.shape[2]
    w1 = w1_ref[...]
    h1 = [jnp.dot(w1, xyz_ref[b], preferred_element_type=jnp.float32)
          + b1_ref[...] for b in range(B)]
    m, rs = _bn_stats(h1, B * M)
    g1 = g1_ref[...]
    bt1 = bt1_ref[...]
    w2 = w2_ref[...]
    scw = scw_ref[...]
    shw = shw_ref[...]
    kw = kw_ref[...]
    for b in range(B):
        hr = jnp.maximum(g1 * (h1[b] - m) * rs + bt1, 0.0)
        geom = jnp.dot(w2, hr, preferred_element_type=jnp.float32) + b2_ref[...]
        sft = sft_ref[b]
        scale = jnp.dot(scw, sft, preferred_element_type=jnp.float32) + scb_ref[...]
        shift = jnp.dot(shw, sft, preferred_element_type=jnp.float32) + shb_ref[...]
        mod = geom * (scale + 1.0) + shift
        k_out_ref[b] = jnp.dot(kw, mod, preferred_element_type=jnp.float32) + kb_ref[...]


def _p2_body(xyz_ref, w1_ref, b1_ref, g1_ref, bt1_ref, w2_ref, b2_ref,
             qw_ref, qb_ref, bdw1_ref, bdb1_ref, bdg1_ref, bdbt1_ref,
             bdw2_ref, bdb2_ref, rpw2_ref, rpb2_ref,
             q_out_ref, qp_out_ref, qb2_out_ref, bdy_out_ref):
    B = xyz_ref.shape[0]
    N = xyz_ref.shape[2]
    w1 = w1_ref[...]
    h1 = [jnp.dot(w1, xyz_ref[b], preferred_element_type=jnp.float32)
          + b1_ref[...] for b in range(B)]
    m, rs = _bn_stats(h1, B * N)
    g1 = g1_ref[...]
    bt1 = bt1_ref[...]
    w2 = w2_ref[...]
    qw = qw_ref[...]
    bdw1 = bdw1_ref[...]
    rpw2 = rpw2_ref[...]
    rpb2 = rpb2_ref[...]
    geoms = []
    hb = []
    for b in range(B):
        hr = jnp.maximum(g1 * (h1[b] - m) * rs + bt1, 0.0)
        geom = jnp.dot(w2, hr, preferred_element_type=jnp.float32) + b2_ref[...]
        geoms.append(geom)
        hb.append(jnp.dot(bdw1, geom, preferred_element_type=jnp.float32)
                  + bdb1_ref[...])
    mb, rsb = _bn_stats(hb, B * N)
    bdg1 = bdg1_ref[...]
    bdbt1 = bdbt1_ref[...]
    bdw2 = bdw2_ref[...]
    for b in range(B):
        q = jnp.dot(qw, geoms[b], preferred_element_type=jnp.float32) + qb_ref[...]
        q_out_ref[b] = q
        qp_out_ref[b] = jax.lax.dot_general(
            rpw2, q, (((0,), (0,)), ((), ())),
            preferred_element_type=jnp.float32)
        qb2_out_ref[b] = jnp.dot(rpb2, q, preferred_element_type=jnp.float32)
        hbr = jnp.maximum(bdg1 * (hb[b] - mb) * rsb + bdbt1, 0.0)
        z = jnp.dot(bdw2, hbr, preferred_element_type=jnp.float32) + bdb2_ref[...]
        bdy_out_ref[b] = 1.0 / (1.0 + jnp.exp(-z))


def _m1_body(a_ref, q_ref, bl_ref, k_ref,
             idx_ref, sv_ref, xg_ref, mom_ref):
    b = pl.program_id(0)
    i = pl.program_id(1)
    BN = a_ref.shape[2]
    M = bl_ref.shape[2]
    a = a_ref[0]
    bl = bl_ref[0]
    q = q_ref[0]
    k = k_ref[0]
    s = jax.lax.dot_general(q, k, (((0,), (0,)), ((), ())),
                            preferred_element_type=jnp.float32)
    ab = jax.lax.dot_general(a, bl, (((0,), (0,)), ((), ())),
                             preferred_element_type=jnp.float32)
    bn2 = jnp.sum(bl * bl, axis=0, keepdims=True)
    d2 = bn2 - 2.0 * ab
    lane = jax.lax.broadcasted_iota(jnp.int32, (BN, M), 1)
    fmax = jnp.float32(3.0e38)
    sels = []
    for _ in range(KNN):
        vmin = jnp.min(d2, axis=1, keepdims=True)
        cand = jnp.where(d2 == vmin, lane, M)
        sel = jnp.min(cand, axis=1, keepdims=True)
        sels.append(sel)
        d2 = jnp.where(lane == sel, fmax, d2)
    idx = jnp.concatenate(sels, axis=1)
    hi = idx >> 7
    lo = idx & 127
    sv = jnp.zeros((BN, KNN), jnp.float32)
    xg = [jnp.zeros((BN, KNN), jnp.float32) for _ in range(3)]
    for c in range(M // 128):
        csel = hi == c
        g = jnp.take_along_axis(s[:, c * 128:(c + 1) * 128], lo, axis=1)
        sv = jnp.where(csel, g, sv)
        for d in range(3):
            bc = jnp.broadcast_to(bl[d:d + 1, c * 128:(c + 1) * 128], (BN, 128))
            gd = jnp.take_along_axis(bc, lo, axis=1)
            xg[d] = jnp.where(csel, gd, xg[d])
    idx_ref[0] = idx.T
    sv_ref[0] = sv.T
    xgt = [x.T for x in xg]
    xg_ref[0] = jnp.concatenate(xgt, axis=0)
    rel = [a[d:d + 1, :] - xgt[d] for d in range(3)]
    base = jnp.zeros((8, 128), jnp.float32)
    r0 = jax.lax.broadcasted_iota(jnp.int32, (8, 128), 0)
    l0 = jax.lax.broadcasted_iota(jnp.int32, (8, 128), 1)
    for d in range(3):
        base = base + jnp.where((r0 == 3) & (l0 == d), jnp.sum(rel[d]), 0.0)
        for e in range(d, 3):
            base = base + jnp.where((r0 == d) & (l0 == e),
                                    jnp.sum(rel[d] * rel[e]), 0.0)

    @pl.when((b == 0) & (i == 0))
    def _():
        mom_ref[...] = jnp.zeros_like(mom_ref)

    mom_ref[...] += base


def _m2_body(a_ref, qp_ref, qb2_ref, sv_ref, xg_ref, idx_ref, val_ref,
             mom_ref, rpw1_ref, rpb1_ref, rpg1_ref, rpbt1_ref, o_ref, *,
             total_count):
    BN = a_ref.shape[2]
    M = val_ref.shape[2]
    mom = mom_ref[...]
    inv_t = 1.0 / total_count
    mu = [jnp.sum(mom[3:4, d:d + 1]) * inv_t for d in range(3)]
    cov = {}
    for d in range(3):
        for e in range(d, 3):
            cov[(d, e)] = jnp.sum(mom[d:d + 1, e:e + 1]) * inv_t - mu[d] * mu[e]
    w1 = rpw1_ref[...]
    cols = [w1[:, d:d + 1] for d in range(3)]
    meanc = rpb1_ref[...]
    for d in range(3):
        meanc = meanc + cols[d] * mu[d]
    varc = jnp.zeros_like(meanc)
    for d in range(3):
        for e in range(3):
            c = cov[(d, e)] if d <= e else cov[(e, d)]
            varc = varc + cols[d] * cols[e] * c
    rs = jax.lax.rsqrt(varc + EPS)
    gscale = rpg1_ref[...] * rs
    w1s = w1 * gscale
    bconst = (rpb1_ref[...] - meanc) * gscale + rpbt1_ref[...]
    a = a_ref[0]
    qp = qp_ref[0]
    xg = xg_ref[0]
    lps = []
    for j in range(KNN):
        rel_j = jnp.concatenate(
            [a[d:d + 1, :] - xg[d * KNN + j:d * KNN + j + 1, :]
             for d in range(3)], axis=0)
        pe_j = jnp.maximum(
            jnp.dot(w1s, rel_j, preferred_element_type=jnp.float32) + bconst,
            0.0)
        lps.append(jnp.sum(qp * pe_j, axis=0, keepdims=True))
    lp = jnp.concatenate(lps, axis=0)
    logits = (sv_ref[0] + lp + qb2_ref[0]) * (1.0 / (128.0 ** 0.5))
    mx = jnp.max(logits, axis=0, keepdims=True)
    e = jnp.exp(logits - mx)
    attn = e / jnp.sum(e, axis=0, keepdims=True)
    idx = idx_ref[0]
    row = jax.lax.broadcasted_iota(jnp.int32, (M, BN), 0)
    amat = jnp.zeros((M, BN), jnp.float32)
    for j in range(KNN):
        amat = amat + jnp.where(row == idx[j:j + 1, :], attn[j:j + 1, :], 0.0)
    o_ref[0] = jnp.dot(val_ref[0], amat, preferred_element_type=jnp.float32)


def kernel(xyz_hr, xyz_lr, sft_feat_lr, val_lr, params):
    p: dict[str, Any] = params
    B, _, N = xyz_hr.shape
    M = xyz_lr.shape[2]
    CV = val_lr.shape[1]
    f32 = jnp.float32

    def col(x):
        return x.reshape(-1, 1)

    k_mat = pl.pallas_call(
        _p1_body,
        out_shape=jax.ShapeDtypeStruct((B, 128, M), f32),
    )(xyz_lr, sft_feat_lr, p['ge_w1'], col(p['ge_b1']), col(p['ge_g1']),
      col(p['ge_beta1']), p['ge_w2'], col(p['ge_b2']), p['sc_w'],
      col(p['sc_b']), p['sh_w'], col(p['sh_b']), p['k_w'], col(p['k_b']))

    q_mat, qp_mat, qb2_mat, bdy = pl.pallas_call(
        _p2_body,
        out_shape=(
            jax.ShapeDtypeStruct((B, 128, N), f32),
            jax.ShapeDtypeStruct((B, 128, N), f32),
            jax.ShapeDtypeStruct((B, 1, N), f32),
            jax.ShapeDtypeStruct((B, 1, N), f32),
        ),
    )(xyz_hr, p['ge_w1'], col(p['ge_b1']), col(p['ge_g1']), col(p['ge_beta1']),
      p['ge_w2'], col(p['ge_b2']), p['q_w'], col(p['q_b']), p['bd_w1'],
      col(p['bd_b1']), col(p['bd_g1']), col(p['bd_beta1']), p['bd_w2'],
      col(p['bd_b2']), p['rp_w2'], p['rp_b2'].reshape(1, -1))

    BN = 256
    grid1 = (B, N // BN)
    idx_a, sv_a, xg_a, mom = pl.pallas_call(
        _m1_body,
        grid=grid1,
        in_specs=[
            pl.BlockSpec((1, 3, BN), lambda b, i: (b, 0, i)),
            pl.BlockSpec((1, 128, BN), lambda b, i: (b, 0, i)),
            pl.BlockSpec((1, 3, M), lambda b, i: (b, 0, 0)),
            pl.BlockSpec((1, 128, M), lambda b, i: (b, 0, 0)),
        ],
        out_specs=(
            pl.BlockSpec((1, KNN, BN), lambda b, i: (b, 0, i)),
            pl.BlockSpec((1, KNN, BN), lambda b, i: (b, 0, i)),
            pl.BlockSpec((1, 3 * KNN, BN), lambda b, i: (b, 0, i)),
            pl.BlockSpec((8, 128), lambda b, i: (0, 0)),
        ),
        out_shape=(
            jax.ShapeDtypeStruct((B, KNN, N), jnp.int32),
            jax.ShapeDtypeStruct((B, KNN, N), f32),
            jax.ShapeDtypeStruct((B, 3 * KNN, N), f32),
            jax.ShapeDtypeStruct((8, 128), f32),
        ),
    )(xyz_hr, q_mat, xyz_lr, k_mat)

    BN2 = 512
    grid2 = (B, N // BN2)
    out = pl.pallas_call(
        functools.partial(_m2_body, total_count=float(B * N * KNN)),
        grid=grid2,
        in_specs=[
            pl.BlockSpec((1, 3, BN2), lambda b, i: (b, 0, i)),
            pl.BlockSpec((1, 128, BN2), lambda b, i: (b, 0, i)),
            pl.BlockSpec((1, 1, BN2), lambda b, i: (b, 0, i)),
            pl.BlockSpec((1, KNN, BN2), lambda b, i: (b, 0, i)),
            pl.BlockSpec((1, 3 * KNN, BN2), lambda b, i: (b, 0, i)),
            pl.BlockSpec((1, KNN, BN2), lambda b, i: (b, 0, i)),
            pl.BlockSpec((1, CV, M), lambda b, i: (b, 0, 0)),
            pl.BlockSpec((8, 128), lambda b, i: (0, 0)),
            pl.BlockSpec((128, 3), lambda b, i: (0, 0)),
            pl.BlockSpec((128, 1), lambda b, i: (0, 0)),
            pl.BlockSpec((128, 1), lambda b, i: (0, 0)),
            pl.BlockSpec((128, 1), lambda b, i: (0, 0)),
        ],
        out_specs=pl.BlockSpec((1, CV, BN2), lambda b, i: (b, 0, i)),
        out_shape=jax.ShapeDtypeStruct((B, CV, N), f32),
    )(xyz_hr, qp_mat, qb2_mat, sv_a, xg_a, idx_a, val_lr, mom,
      p['rp_w1'], col(p['rp_b1']), col(p['rp_g1']), col(p['rp_beta1']))

    return out, bdy
